# baseline (speedup 1.0000x reference)
import jax
import jax.numpy as jnp
from jax import lax
from jax.experimental import pallas as pl
from jax.experimental.pallas import tpu as pltpu
from jax.experimental.pallas import tpu_sc as plsc

NUM_CORES = 2
NUM_SUBCORES = 16
NW = NUM_CORES * NUM_SUBCORES

D = 64
RPW = 512
RPC = 16
CPW = RPW // RPC
NB = 3


def _gather_kernel(lab_hbm, table_hbm, out_hbm,
                   idx_v, rows0, rows1, rows2,
                   sg0, sg1, sg2, so0, so1, so2):
    W = lab_hbm.shape[1]
    rows = (rows0, rows1, rows2)
    sg = (sg0, sg1, sg2)
    so = (so0, so1, so2)
    wid = lax.axis_index("s") * NUM_CORES + lax.axis_index("c")
    r_base = wid * RPW

    def start_gather(c, b):
        for k in range(RPC):
            pltpu.async_copy(
                table_hbm.at[idx_v.at[c * RPC + k]], rows[b].at[k], sg[b])

    def wait_gather(c, b):
        for k in range(RPC):
            pltpu.make_async_copy(
                table_hbm.at[idx_v.at[c * RPC + k]], rows[b].at[k], sg[b]
            ).wait()

    def start_out(c, b):
        pltpu.async_copy(
            rows[b], out_hbm.at[pl.ds(r_base + c * RPC, RPC)], so[b])

    def wait_out(b):
        pltpu.make_async_copy(
            rows[b], out_hbm.at[pl.ds(r_base, RPC)], so[b]).wait()

    pltpu.sync_copy(lab_hbm.at[pl.ds(r_base, RPW)], idx_v)

    start_gather(0, 0)
    start_gather(1, 1)
    wait_gather(0, 0); start_out(0, 0); start_gather(2, 2)
    wait_gather(1, 1); start_out(1, 1); wait_out(0); start_gather(3, 0)
    wait_gather(2, 2); start_out(2, 2); wait_out(1); start_gather(4, 1)

    def body(i, _):
        for b in range(NB):
            c = NB * i + b
            wait_gather(c, b)
            start_out(c, b)
            wait_out((b + 2) % NB)
            start_gather(c + 2, (b + 2) % NB)
        return _

    lax.fori_loop(1, (CPW - 2) // NB, body, None)

    c0, c1 = CPW - 2, CPW - 1
    b0, b1 = c0 % NB, c1 % NB
    wait_gather(c0, b0); start_out(c0, b0)
    wait_gather(c1, b1); start_out(c1, b1)
    wait_out((b1 + 1) % NB); wait_out(b0); wait_out(b1)


def kernel(labels, label_embed_weight):
    B0, B1 = labels.shape
    assert B0 == NW * RPW and CPW % NB == 2 % NB

    run = pl.kernel(
        _gather_kernel,
        out_type=jax.ShapeDtypeStruct((B0, B1, D), jnp.float32),
        mesh=plsc.VectorSubcoreMesh(
            core_axis_name="c", subcore_axis_name="s",
            num_cores=NUM_CORES, num_subcores=NUM_SUBCORES,
        ),
        scratch_types=[
            pltpu.VMEM((RPW, B1), jnp.int32),
            pltpu.VMEM((RPC, B1, D), jnp.float32),
            pltpu.VMEM((RPC, B1, D), jnp.float32),
            pltpu.VMEM((RPC, B1, D), jnp.float32),
            pltpu.SemaphoreType.DMA,
            pltpu.SemaphoreType.DMA,
            pltpu.SemaphoreType.DMA,
            pltpu.SemaphoreType.DMA,
            pltpu.SemaphoreType.DMA,
            pltpu.SemaphoreType.DMA,
        ],
        compiler_params=pltpu.CompilerParams(use_tc_tiling_on_sc=False),
    )
    return run(labels, label_embed_weight)
